# Initial kernel scaffold; baseline (speedup 1.0000x reference)
import jax
import jax.numpy as jnp
from jax.experimental import pallas as pl


def kernel(x, edge_index, W_l0, b_l0, W_r0, W_l1, b_l1, W_r1, W_l2, b_l2, W_r2):
    raise NotImplementedError("write your pallas kernel here")


# speedup vs baseline: 6.9402x; 6.9402x over previous
import functools

import jax
import jax.numpy as jnp
from jax import lax
from jax.experimental import pallas as pl
from jax.experimental.pallas import tpu as pltpu
from jax.experimental.pallas import tpu_sc as plsc

N = 10000
E = 320000
D = 128

NC = 2
NS = 16
NW = NC * NS
EW = E // NW
CH = 80
NCHUNK = EW // CH
SB = 25
NSB = NCHUNK // SB
NP = 10240
RPS = NP // NS


def _fill_2d(ref, rows, width, value):
    def row(i, _):
        def col(j, _):
            ref[i, pl.ds(j * 16, 16)] = jnp.full((16,), value, jnp.float32)
            return 0
        lax.fori_loop(0, width // 16, col, 0)
        return 0
    lax.fori_loop(0, rows, row, 0)


def _zero_own_slice(rows_v, acc, s):
    _fill_2d(rows_v, CH, D, 0.0)
    for k in range(RPS // CH):
        pltpu.sync_copy(rows_v, acc.at[pl.ds(s * RPS + k * CH, CH)])


def _copy_out(acc, out_hbm, c, s):
    pltpu.sync_copy(acc.at[pl.ds(s * RPS, RPS)],
                    out_hbm.at[c, pl.ds(s * RPS, RPS)])


def _sc_agg_body(y_hbm, src_hbm, dst_hbm, out_hbm, src_v, dst_v, rows_v,
                 acc, sem):
    c = lax.axis_index("c")
    s = lax.axis_index("s")
    wid = c * NS + s

    _zero_own_slice(rows_v, acc, s)
    plsc.subcore_barrier()

    def superchunk(sb, _):
        pltpu.sync_copy(src_hbm.at[wid, sb], src_v)
        pltpu.sync_copy(dst_hbm.at[wid, sb], dst_v)

        def chunk(j, _):
            pltpu.async_copy(y_hbm.at[src_v.at[j]], rows_v, sem).wait()
            pltpu.sync_copy(rows_v, acc.at[dst_v.at[j]], add=True)
            return 0
        lax.fori_loop(0, SB, chunk, 0)
        return 0
    lax.fori_loop(0, NSB, superchunk, 0)

    plsc.subcore_barrier()
    _copy_out(acc, out_hbm, c, s)


def _sc_cnt_body(dst_hbm, out_hbm, dst_v, rows_v, acc, sem):
    c = lax.axis_index("c")
    s = lax.axis_index("s")
    wid = c * NS + s

    _zero_own_slice(rows_v, acc, s)
    plsc.subcore_barrier()

    _fill_2d(rows_v, CH, D, 1.0)

    def superchunk(sb, _):
        pltpu.sync_copy(dst_hbm.at[wid, sb], dst_v)

        def chunk(j, _):
            pltpu.sync_copy(rows_v, acc.at[dst_v.at[j]], add=True)
            return 0
        lax.fori_loop(0, SB, chunk, 0)
        return 0
    lax.fori_loop(0, NSB, superchunk, 0)

    plsc.subcore_barrier()
    _copy_out(acc, out_hbm, c, s)


@functools.lru_cache(maxsize=None)
def _make_sc_agg():
    mesh = plsc.VectorSubcoreMesh(core_axis_name="c", subcore_axis_name="s",
                                  num_cores=NC, num_subcores=NS)
    return pl.kernel(
        _sc_agg_body,
        out_type=[jax.ShapeDtypeStruct((NC, NP, D), jnp.float32)],
        mesh=mesh,
        scratch_types=[
            pltpu.VMEM((SB, CH), jnp.int32),
            pltpu.VMEM((SB, CH), jnp.int32),
            pltpu.VMEM((CH, D), jnp.float32),
            pltpu.VMEM_SHARED((NP, D), jnp.float32),
            pltpu.SemaphoreType.DMA,
        ],
    )


@functools.lru_cache(maxsize=None)
def _make_sc_cnt():
    mesh = plsc.VectorSubcoreMesh(core_axis_name="c", subcore_axis_name="s",
                                  num_cores=NC, num_subcores=NS)
    return pl.kernel(
        _sc_cnt_body,
        out_type=[jax.ShapeDtypeStruct((NC, NP, D), jnp.float32)],
        mesh=mesh,
        scratch_types=[
            pltpu.VMEM((SB, CH), jnp.int32),
            pltpu.VMEM((CH, D), jnp.float32),
            pltpu.VMEM_SHARED((NP, D), jnp.float32),
            pltpu.SemaphoreType.DMA,
        ],
    )


RB = 1000


def _tc_combine_body(relu, acc_ref, cnt_ref, h_ref, wl_ref, b_ref, wr_ref,
                     out_ref):
    a = acc_ref[0] + acc_ref[1]
    cnt = cnt_ref[0, :, 0:1] + cnt_ref[1, :, 0:1]
    inv = 1.0 / jnp.maximum(cnt, 1.0)
    m = a * inv
    out = (jnp.dot(m, wl_ref[...], preferred_element_type=jnp.float32)
           + b_ref[...]
           + jnp.dot(h_ref[...], wr_ref[...],
                     preferred_element_type=jnp.float32))
    if relu:
        out = jnp.maximum(out, 0.0)
    out_ref[...] = out


@functools.lru_cache(maxsize=None)
def _make_combine(relu):
    return pl.pallas_call(
        functools.partial(_tc_combine_body, relu),
        grid=(N // RB,),
        in_specs=[
            pl.BlockSpec((NC, RB, D), lambda i: (0, i, 0)),
            pl.BlockSpec((NC, RB, D), lambda i: (0, i, 0)),
            pl.BlockSpec((RB, D), lambda i: (i, 0)),
            pl.BlockSpec((D, D), lambda i: (0, 0)),
            pl.BlockSpec((1, D), lambda i: (0, 0)),
            pl.BlockSpec((D, D), lambda i: (0, 0)),
        ],
        out_specs=pl.BlockSpec((RB, D), lambda i: (i, 0)),
        out_shape=jax.ShapeDtypeStruct((N, D), jnp.float32),
    )


def kernel(x, edge_index, W_l0, b_l0, W_r0, W_l1, b_l1, W_r1, W_l2, b_l2,
           W_r2):
    src = edge_index[0].reshape(NW, NSB, SB, CH)
    dst = edge_index[1].reshape(NW, NSB, SB, CH)
    sc_agg, sc_cnt = _make_sc_agg(), _make_sc_cnt()
    combine_relu, combine_last = _make_combine(True), _make_combine(False)

    (cnt2,) = sc_cnt(dst)
    (acc2,) = sc_agg(x, src, dst)
    h1 = combine_relu(acc2, cnt2, x, W_l0, b_l0.reshape(1, D), W_r0)
    (acc2,) = sc_agg(h1, src, dst)
    h2 = combine_relu(acc2, cnt2, h1, W_l1, b_l1.reshape(1, D), W_r1)
    (acc2,) = sc_agg(h2, src, dst)
    return combine_last(acc2, cnt2, h2, W_l2, b_l2.reshape(1, D), W_r2)
